# Initial kernel scaffold; baseline (speedup 1.0000x reference)
import jax
import jax.numpy as jnp
from jax.experimental import pallas as pl


def kernel(xyz, table, voxel_idx_map, voxel_size, voxel_offset):
    raise NotImplementedError("write your pallas kernel here")


# speedup vs baseline: 2.0059x; 2.0059x over previous
import functools

import jax
import jax.numpy as jnp
from jax import lax
from jax.experimental import pallas as pl
from jax.experimental.pallas import tpu as pltpu
from jax.experimental.pallas import tpu_sc as plsc

CH = 64
N_FREQS = 4
N_FREQS_XYZ = 10
GRID = (160, 160, 160)

NC, NS, L = 2, 16, 16
NW = NC * NS

C = 128
G = C // L
NROW = 8 * C
NIR = NROW // 128


def _sc_lookup(px, py, pz, map_flat, table):
    B = px.shape[0]
    P = B // NW
    nch = P // C
    gx, gy, gz = GRID

    mesh = plsc.VectorSubcoreMesh(core_axis_name="c", subcore_axis_name="s")

    @functools.partial(
        pl.kernel,
        out_type=[
            jax.ShapeDtypeStruct((B, CH), jnp.float32),
            jax.ShapeDtypeStruct((B,), jnp.int32),
        ],
        mesh=mesh,
        scratch_types=[
            pltpu.VMEM((C,), jnp.float32),
            pltpu.VMEM((C,), jnp.float32),
            pltpu.VMEM((C,), jnp.float32),
            pltpu.VMEM((NIR, 128), jnp.int32),
            pltpu.VMEM((NIR, 128), jnp.int32),
            pltpu.VMEM((NROW,), jnp.float32),
            pltpu.VMEM((NIR, 128), jnp.int32),
            pltpu.VMEM((NROW, CH), jnp.float32),
            pltpu.VMEM((C, CH), jnp.float32),
            pltpu.VMEM((C,), jnp.int32),
            pltpu.SemaphoreType.DMA,
        ],
        compiler_params=pltpu.CompilerParams(
            needs_layout_passes=False, use_tc_tiling_on_sc=False),
    )
    def body(px_h, py_h, pz_h, map_h, tab_h, feat_h, mask_h,
             xv, yv, zv, fidx, mval, wbuf, tidx, rows, featb, maskb, sem):
        wid = lax.axis_index("s") * NC + lax.axis_index("c")
        iota = lax.iota(jnp.int32, L)

        def chunk_body(ci, _):
            start = wid * P + ci * C
            pltpu.sync_copy(px_h.at[pl.ds(start, C)], xv)
            pltpu.sync_copy(py_h.at[pl.ds(start, C)], yv)
            pltpu.sync_copy(pz_h.at[pl.ds(start, C)], zv)

            for g in range(G):
                x = xv[pl.ds(g * L, L)]
                y = yv[pl.ds(g * L, L)]
                z = zv[pl.ds(g * L, L)]
                bx = x.astype(jnp.int32)
                by = y.astype(jnp.int32)
                bz = z.astype(jnp.int32)
                fx = x - bx.astype(jnp.float32)
                fy = y - by.astype(jnp.float32)
                fz = z - bz.astype(jnp.float32)
                ix = (jnp.minimum(jnp.maximum(bx, 0), gx - 1),
                      jnp.minimum(jnp.maximum(bx + 1, 0), gx - 1))
                iy = (jnp.minimum(jnp.maximum(by, 0), gy - 1),
                      jnp.minimum(jnp.maximum(by + 1, 0), gy - 1))
                iz = (jnp.minimum(jnp.maximum(bz, 0), gz - 1),
                      jnp.minimum(jnp.maximum(bz + 1, 0), gz - 1))
                wx = (1.0 - fx, fx)
                wy = (1.0 - fy, fy)
                wz = (1.0 - fz, fz)
                for dx in (0, 1):
                    rx = ix[dx] * (gy * gz)
                    for dy in (0, 1):
                        ry = rx + iy[dy] * gz
                        for dz in (0, 1):
                            k = dx * 4 + dy * 2 + dz
                            fidx[g, pl.ds(k * L, L)] = ry + iz[dz]
                            wbuf[pl.ds(g * 128 + k * L, L)] = (
                                wx[dx] * wy[dy] * wz[dz])

            descs = [pltpu.async_copy(map_h.at[fidx.at[j]], mval.at[j], sem)
                     for j in range(NIR)]
            for d in descs:
                d.wait()

            for g in range(G):
                macc = jnp.zeros((L,), jnp.int32)
                for k in range(8):
                    v = mval[g, pl.ds(k * L, L)]
                    valid = v >= 0
                    w = wbuf[pl.ds(g * 128 + k * L, L)]
                    wbuf[pl.ds(g * 128 + k * L, L)] = jnp.where(valid, w, 0.0)
                    tidx[g, pl.ds(k * L, L)] = jnp.maximum(v, 0)
                    macc = jnp.where(valid, 1, macc)
                maskb[pl.ds(g * L, L)] = macc

            descs = [pltpu.async_copy(tab_h.at[tidx.at[j]],
                                      rows.at[pl.ds(j * 128, 128)], sem)
                     for j in range(NIR)]
            for d in descs:
                d.wait()

            for g in range(G):
                ws = [wbuf[pl.ds(g * 128 + k * L, L)] for k in range(8)]
                ridx = [g * 128 + k * L + iota for k in range(8)]
                pidx = g * L + iota

                def ch_body(ch, _, ws=ws, ridx=ridx, pidx=pidx):
                    chv = jnp.full((L,), ch, jnp.int32)
                    acc = ws[0] * plsc.load_gather(rows, [ridx[0], chv])
                    for k in range(1, 8):
                        acc += ws[k] * plsc.load_gather(rows, [ridx[k], chv])
                    plsc.store_scatter(featb, [pidx, chv], acc)
                    return _

                lax.fori_loop(0, CH, ch_body, 0)

            pltpu.sync_copy(featb, feat_h.at[pl.ds(start, C)])
            pltpu.sync_copy(maskb, mask_h.at[pl.ds(start, C)])
            return _

        lax.fori_loop(0, nch, chunk_body, 0)

    return body(px, py, pz, map_flat, table)


def _pe_body(feat_ref, xyz_ref, out_ref):
    f = feat_ref[...]
    s1 = jnp.sin(f)
    c1 = jnp.cos(f)
    s2 = 2.0 * s1 * c1
    c2 = 1.0 - 2.0 * s1 * s1
    s4 = 2.0 * s2 * c2
    c4 = 1.0 - 2.0 * s2 * s2
    s8 = 2.0 * s4 * c4
    c8 = 1.0 - 2.0 * s4 * s4
    x = xyz_ref[...]
    parts = [f, s1, c1, s2, c2, s4, c4, s8, c8, x]
    s = jnp.sin(x)
    c = jnp.cos(x)
    for k in range(N_FREQS_XYZ):
        parts.append(s)
        parts.append(c)
        if k < N_FREQS_XYZ - 1:
            s, c = 2.0 * s * c, 1.0 - 2.0 * s * s
    out_ref[...] = jnp.concatenate(parts, axis=1)


def _pe_call(feat, xyz):
    B = feat.shape[0]
    R = 512
    dout = CH * (2 * N_FREQS + 1) + 3 * (2 * N_FREQS_XYZ + 1)
    return pl.pallas_call(
        _pe_body,
        grid=(B // R,),
        in_specs=[
            pl.BlockSpec((R, CH), lambda i: (i, 0)),
            pl.BlockSpec((R, 3), lambda i: (i, 0)),
        ],
        out_specs=pl.BlockSpec((R, dout), lambda i: (i, 0)),
        out_shape=jax.ShapeDtypeStruct((B, dout), jnp.float32),
    )(feat, xyz)


def kernel(xyz, table, voxel_idx_map, voxel_size, voxel_offset):
    pos = (xyz + voxel_offset) / voxel_size
    post = pos.T
    map_flat = voxel_idx_map.reshape(-1)
    feat, mask_i = _sc_lookup(post[0], post[1], post[2], map_flat, table)
    out = _pe_call(feat, xyz)
    return out, mask_i != 0
